# Initial kernel scaffold; baseline (speedup 1.0000x reference)
import jax
import jax.numpy as jnp
from jax.experimental import pallas as pl


def kernel(x, edges, W1, b1, W2, b2):
    raise NotImplementedError("write your pallas kernel here")


# speedup vs baseline: 1.0014x; 1.0014x over previous
import functools

import jax
import jax.numpy as jnp
from jax.experimental import pallas as pl
from jax.experimental.pallas import tpu as pltpu

N = 50000
FEATS = 128
HIDDEN = 64
CLASSES = 64
ALPHA = 0.1
DEPTH = 10

MLP_BLK = 1000


def _mlp_body(x_ref, w1_ref, b1_ref, w2_ref, b2_ref, out_ref):
    h = jnp.maximum(
        jnp.dot(x_ref[...], w1_ref[...], preferred_element_type=jnp.float32)
        + b1_ref[...],
        0.0,
    )
    out_ref[...] = (
        jnp.dot(h, w2_ref[...], preferred_element_type=jnp.float32) + b2_ref[...]
    )


def _mlp(x, W1, b1, W2, b2):
    grid = N // MLP_BLK
    return pl.pallas_call(
        _mlp_body,
        grid=(grid,),
        in_specs=[
            pl.BlockSpec((MLP_BLK, FEATS), lambda i: (i, 0)),
            pl.BlockSpec((FEATS, HIDDEN), lambda i: (0, 0)),
            pl.BlockSpec((1, HIDDEN), lambda i: (0, 0)),
            pl.BlockSpec((HIDDEN, CLASSES), lambda i: (0, 0)),
            pl.BlockSpec((1, CLASSES), lambda i: (0, 0)),
        ],
        out_specs=pl.BlockSpec((MLP_BLK, CLASSES), lambda i: (i, 0)),
        out_shape=jax.ShapeDtypeStruct((N, CLASSES), jnp.float32),
    )(x, W1, b1.reshape(1, HIDDEN), W2, b2.reshape(1, CLASSES))


def kernel(x, edges, W1, b1, W2, b2):
    src = edges[0]
    dst = edges[1]
    deg = jnp.zeros((N,), dtype=jnp.float32).at[dst].add(1.0)
    deg = deg + jnp.zeros((N,), dtype=jnp.float32).at[src].add(1.0)
    deg = jnp.maximum(deg, 1.0)
    dinv = jax.lax.rsqrt(deg)
    edge_w = dinv[src] * dinv[dst]

    h0 = _mlp(x, W1, b1, W2, b2)
    h = h0
    for _ in range(DEPTH):
        msg = h[src] * edge_w[:, None]
        agg = jnp.zeros_like(h).at[dst].add(msg)
        h = agg * (1.0 - ALPHA) + ALPHA * h0
    return h


# speedup vs baseline: 9.7661x; 9.7527x over previous
import functools

import jax
import jax.numpy as jnp
from jax import lax
from jax.experimental import pallas as pl
from jax.experimental.pallas import tpu as pltpu
from jax.experimental.pallas import tpu_sc as plsc

N = 50000
FEATS = 128
HIDDEN = 64
CLASSES = 64
ALPHA = 0.1
DEPTH = 10

E = 800000
EP = 802816
EPR = EP // 128
ROWS_PER_TILE = EPR // 16
BROWS = 4
BLOCKS_PER_TILE = ROWS_PER_TILE // BROWS
DEG_BROWS = 8
DEG_BLOCKS = ROWS_PER_TILE // DEG_BROWS

NPAD = 50176
ZROWS = NPAD // 16
WROWS = 3128
NOUT = 16 * WROWS

MLP_BLK = 1000
UPD_BLK = 2000

_MESH = plsc.VectorSubcoreMesh(core_axis_name="c", subcore_axis_name="s")
_SC_PARAMS = pltpu.CompilerParams(use_tc_tiling_on_sc=False)


@functools.partial(
    pl.kernel,
    out_type=jax.ShapeDtypeStruct((2, NOUT, 32), jnp.float32),
    mesh=_MESH,
    scratch_types=[
        pltpu.VMEM_SHARED((NPAD, 32), jnp.float32),
        pltpu.VMEM((128, 32), jnp.float32),
        pltpu.VMEM((8, 128), jnp.int32),
        pltpu.SemaphoreType.DMA,
    ],
    compiler_params=_SC_PARAMS,
)
def _deg_sc(ep_hbm, ones_hbm, zeros_hbm, sdeg_hbm, acc, onesv, idxv, sem):
    c = lax.axis_index("c")
    t = lax.axis_index("s")
    pltpu.sync_copy(zeros_hbm, acc.at[pl.ds(t * ZROWS, ZROWS)])
    pltpu.sync_copy(ones_hbm, onesv)
    plsc.subcore_barrier()

    def blk(i, carry):
        bb = t * ROWS_PER_TILE + i * DEG_BROWS
        pltpu.sync_copy(ep_hbm.at[c, pl.ds(bb, DEG_BROWS)], idxv)
        cps = [
            pltpu.async_copy(onesv, acc.at[idxv.at[j]], sem, add=True)
            for j in range(DEG_BROWS)
        ]
        for d in cps:
            d.wait()
        return carry

    lax.fori_loop(0, DEG_BLOCKS, blk, 0)
    plsc.subcore_barrier()
    rb = t * WROWS
    pltpu.sync_copy(acc.at[pl.ds(rb, WROWS)], sdeg_hbm.at[c, pl.ds(rb, WROWS)])


@functools.partial(
    pl.kernel,
    out_type=jax.ShapeDtypeStruct((2, NOUT, 32), jnp.float32),
    mesh=_MESH,
    scratch_types=[
        pltpu.VMEM_SHARED((NPAD, 32), jnp.float32),
        pltpu.VMEM((BROWS, 128), jnp.int32),
        pltpu.VMEM((BROWS, 128), jnp.int32),
        pltpu.VMEM((BROWS * 128, 32), jnp.float32),
        pltpu.SemaphoreType.DMA,
        pltpu.SemaphoreType.DMA,
    ],
    compiler_params=_SC_PARAMS,
)
def _prop_sc(p2_hbm, gsrc_hbm, dst_hbm, zeros_hbm, s_hbm,
             acc, gidx, didx, rows, sem_g, sem_s):
    c = lax.axis_index("c")
    t = lax.axis_index("s")
    pltpu.sync_copy(zeros_hbm, acc.at[pl.ds(t * ZROWS, ZROWS)])
    plsc.subcore_barrier()

    def blk(i, carry):
        bb = t * ROWS_PER_TILE + i * BROWS
        pltpu.sync_copy(gsrc_hbm.at[c, pl.ds(bb, BROWS)], gidx)
        pltpu.sync_copy(dst_hbm.at[pl.ds(bb, BROWS)], didx)
        gs = [
            pltpu.async_copy(p2_hbm.at[gidx.at[j]],
                             rows.at[pl.ds(j * 128, 128)], sem_g)
            for j in range(BROWS)
        ]
        for d in gs:
            d.wait()
        ss = [
            pltpu.async_copy(rows.at[pl.ds(j * 128, 128)],
                             acc.at[didx.at[j]], sem_s, add=True)
            for j in range(BROWS)
        ]
        for d in ss:
            d.wait()
        return carry

    lax.fori_loop(0, BLOCKS_PER_TILE, blk, 0)
    plsc.subcore_barrier()
    rb = t * WROWS
    pltpu.sync_copy(acc.at[pl.ds(rb, WROWS)], s_hbm.at[c, pl.ds(rb, WROWS)])


def _mlp_body(x_ref, w1_ref, b1_ref, w2_ref, b2_ref, sd_ref,
              h0_ref, p0_ref, c1_ref, c1f_ref):
    deg = jnp.maximum(sd_ref[0, :, 0:1] + sd_ref[1, :, 0:1], 1.0)
    dinv = lax.rsqrt(deg)
    h = jnp.maximum(
        jnp.dot(x_ref[...], w1_ref[...], preferred_element_type=jnp.float32)
        + b1_ref[...],
        0.0,
    )
    h0 = jnp.dot(h, w2_ref[...], preferred_element_type=jnp.float32) + b2_ref[...]
    h0_ref[...] = h0
    p0_ref[...] = dinv * h0
    c1_ref[...] = 0.9 / deg
    c1f_ref[...] = 0.9 * dinv


def _mlp_prep(x, W1, b1, W2, b2, sdeg):
    grid = N // MLP_BLK
    return pl.pallas_call(
        _mlp_body,
        grid=(grid,),
        in_specs=[
            pl.BlockSpec((MLP_BLK, FEATS), lambda i: (i, 0)),
            pl.BlockSpec((FEATS, HIDDEN), lambda i: (0, 0)),
            pl.BlockSpec((1, HIDDEN), lambda i: (0, 0)),
            pl.BlockSpec((HIDDEN, CLASSES), lambda i: (0, 0)),
            pl.BlockSpec((1, CLASSES), lambda i: (0, 0)),
            pl.BlockSpec((2, MLP_BLK, 32), lambda i: (0, i, 0)),
        ],
        out_specs=[
            pl.BlockSpec((MLP_BLK, CLASSES), lambda i: (i, 0)),
            pl.BlockSpec((MLP_BLK, CLASSES), lambda i: (i, 0)),
            pl.BlockSpec((MLP_BLK, 1), lambda i: (i, 0)),
            pl.BlockSpec((MLP_BLK, 1), lambda i: (i, 0)),
        ],
        out_shape=[
            jax.ShapeDtypeStruct((N, CLASSES), jnp.float32),
            jax.ShapeDtypeStruct((N, CLASSES), jnp.float32),
            jax.ShapeDtypeStruct((N, 1), jnp.float32),
            jax.ShapeDtypeStruct((N, 1), jnp.float32),
        ],
    )(x, W1, b1.reshape(1, HIDDEN), W2, b2.reshape(1, CLASSES), sdeg)


def _upd_body(s_ref, base_ref, coef_ref, out_ref):
    cf = coef_ref[...]
    out_ref[...] = (
        jnp.concatenate([cf * s_ref[0], cf * s_ref[1]], axis=1)
        + ALPHA * base_ref[...]
    )


def _upd(s, base, coef):
    grid = N // UPD_BLK
    return pl.pallas_call(
        _upd_body,
        grid=(grid,),
        in_specs=[
            pl.BlockSpec((2, UPD_BLK, 32), lambda i: (0, i, 0)),
            pl.BlockSpec((UPD_BLK, CLASSES), lambda i: (i, 0)),
            pl.BlockSpec((UPD_BLK, 1), lambda i: (i, 0)),
        ],
        out_specs=pl.BlockSpec((UPD_BLK, CLASSES), lambda i: (i, 0)),
        out_shape=jax.ShapeDtypeStruct((N, CLASSES), jnp.float32),
    )(s, base, coef)


def kernel(x, edges, W1, b1, W2, b2):
    src = edges[0]
    dst = edges[1]
    padn = jnp.full((EP - E,), N, dtype=jnp.int32)
    ep = jnp.stack([
        jnp.concatenate([src, padn]),
        jnp.concatenate([dst, padn]),
    ]).reshape(2, EPR, 128)
    src_p = jnp.concatenate([src, jnp.zeros((EP - E,), jnp.int32)])
    gsrc = jnp.stack([2 * src_p, 2 * src_p + 1]).reshape(2, EPR, 128)
    dst_p = jnp.concatenate([dst, padn]).reshape(EPR, 128)

    ones = jnp.ones((128, 32), jnp.float32)
    zeros = jnp.zeros((ZROWS, 32), jnp.float32)

    sdeg = _deg_sc(ep, ones, zeros)
    h0, p0, c1, c1f = _mlp_prep(x, W1, b1, W2, b2, sdeg)

    p = p0
    for it in range(DEPTH):
        s = _prop_sc(p.reshape(2 * N, 32), gsrc, dst_p, zeros)
        if it < DEPTH - 1:
            p = _upd(s, p0, c1)
        else:
            p = _upd(s, h0, c1f)
    return p


# speedup vs baseline: 11.9169x; 1.2202x over previous
import functools

import jax
import jax.numpy as jnp
from jax import lax
from jax.experimental import pallas as pl
from jax.experimental.pallas import tpu as pltpu
from jax.experimental.pallas import tpu_sc as plsc

N = 50000
FEATS = 128
HIDDEN = 64
CLASSES = 64
ALPHA = 0.1
DEPTH = 10

E = 800000
EP = 802816
EPR = EP // 128
ROWS_PER_TILE = EPR // 16
DEG_BROWS = 8
DEG_BLOCKS = ROWS_PER_TILE // DEG_BROWS

EPP = 804864
EPRP = EPP // 128
PROWS = EPRP // 16
PB = 3
PBLOCKS = PROWS // PB
CIDX_ROWS = EPRP + 8

NPAD = 50048
ZROWS = NPAD // 16
WROWS = 3128
NOUT = 16 * WROWS

MLP_BLK = 1000
UPD_BLK = 2000

_MESH = plsc.VectorSubcoreMesh(core_axis_name="c", subcore_axis_name="s")
_SC_PARAMS = pltpu.CompilerParams(use_tc_tiling_on_sc=False)


@functools.partial(
    pl.kernel,
    out_type=jax.ShapeDtypeStruct((2, NOUT, 32), jnp.float32),
    mesh=_MESH,
    scratch_types=[
        pltpu.VMEM_SHARED((NPAD, 32), jnp.float32),
        pltpu.VMEM((128, 32), jnp.float32),
        pltpu.VMEM((8, 128), jnp.int32),
        pltpu.SemaphoreType.DMA,
    ],
    compiler_params=_SC_PARAMS,
)
def _deg_sc(ep_hbm, ones_hbm, zeros_hbm, sdeg_hbm, acc, onesv, idxv, sem):
    c = lax.axis_index("c")
    t = lax.axis_index("s")
    pltpu.sync_copy(zeros_hbm, acc.at[pl.ds(t * ZROWS, ZROWS)])
    pltpu.sync_copy(ones_hbm, onesv)
    plsc.subcore_barrier()

    def blk(i, carry):
        bb = t * ROWS_PER_TILE + i * DEG_BROWS
        pltpu.sync_copy(ep_hbm.at[c, pl.ds(bb, DEG_BROWS)], idxv)
        cps = [
            pltpu.async_copy(onesv, acc.at[idxv.at[j]], sem, add=True)
            for j in range(DEG_BROWS)
        ]
        for d in cps:
            d.wait()
        return carry

    lax.fori_loop(0, DEG_BLOCKS, blk, 0)
    plsc.subcore_barrier()
    rb = t * WROWS
    pltpu.sync_copy(acc.at[pl.ds(rb, WROWS)], sdeg_hbm.at[c, pl.ds(rb, WROWS)])


@functools.partial(
    pl.kernel,
    out_type=jax.ShapeDtypeStruct((2, NOUT, 32), jnp.float32),
    mesh=_MESH,
    scratch_types=[
        pltpu.VMEM_SHARED((NPAD, 32), jnp.float32),
        pltpu.VMEM((3, PB, 2, 128), jnp.int32),
        pltpu.VMEM((2, PB * 128, 32), jnp.float32),
        pltpu.SemaphoreType.DMA,
        pltpu.SemaphoreType.DMA,
        pltpu.SemaphoreType.DMA,
    ],
    compiler_params=_SC_PARAMS,
)
def _prop_sc(p2_hbm, cidx_hbm, zeros_hbm, s_hbm,
             acc, ib, rows, sem_i, sem_g, sem_s):
    c = lax.axis_index("c")
    t = lax.axis_index("s")
    pltpu.sync_copy(zeros_hbm, acc.at[pl.ds(t * ZROWS, ZROWS)])
    base = t * PROWS
    pltpu.sync_copy(cidx_hbm.at[c, pl.ds(base, PB)], ib.at[0])
    plsc.subcore_barrier()
    for j in range(PB):
        pltpu.async_copy(p2_hbm.at[ib.at[0, j, 0]],
                         rows.at[0, pl.ds(j * 128, 128)], sem_g)

    def blk(k, carry):
        cur3 = k % 3
        nxt3 = (k + 1) % 3
        cur2 = k % 2
        nxt2 = (k + 1) % 2
        pltpu.async_copy(cidx_hbm.at[c, pl.ds(base + (k + 1) * PB, PB)],
                         ib.at[nxt3], sem_i)
        for j in range(PB):
            pltpu.make_async_copy(p2_hbm.at[ib.at[cur3, j, 0]],
                                  rows.at[cur2, pl.ds(j * 128, 128)],
                                  sem_g).wait()
        for j in range(PB):
            pltpu.async_copy(rows.at[cur2, pl.ds(j * 128, 128)],
                             acc.at[ib.at[cur3, j, 1]], sem_s, add=True)

        @pl.when(k > 0)
        def _():
            for j in range(PB):
                pltpu.make_async_copy(rows.at[nxt2, pl.ds(j * 128, 128)],
                                      acc.at[ib.at[cur3, j, 1]], sem_s).wait()

        pltpu.make_async_copy(cidx_hbm.at[c, pl.ds(base, PB)],
                              ib.at[nxt3], sem_i).wait()

        @pl.when(k < PBLOCKS - 1)
        def _():
            for j in range(PB):
                pltpu.async_copy(p2_hbm.at[ib.at[nxt3, j, 0]],
                                 rows.at[nxt2, pl.ds(j * 128, 128)], sem_g)

        return carry

    lax.fori_loop(0, PBLOCKS, blk, 0)
    for j in range(PB):
        pltpu.make_async_copy(rows.at[0, pl.ds(j * 128, 128)],
                              acc.at[ib.at[1, j, 1]], sem_s).wait()
    plsc.subcore_barrier()
    rb = t * WROWS
    pltpu.sync_copy(acc.at[pl.ds(rb, WROWS)], s_hbm.at[c, pl.ds(rb, WROWS)])


def _mlp_body(x_ref, w1_ref, b1_ref, w2_ref, b2_ref, sd_ref,
              h0_ref, p0_ref, c1_ref, c1f_ref):
    deg = jnp.maximum(sd_ref[0, :, 0:1] + sd_ref[1, :, 0:1], 1.0)
    dinv = lax.rsqrt(deg)
    h = jnp.maximum(
        jnp.dot(x_ref[...], w1_ref[...], preferred_element_type=jnp.float32)
        + b1_ref[...],
        0.0,
    )
    h0 = jnp.dot(h, w2_ref[...], preferred_element_type=jnp.float32) + b2_ref[...]
    h0_ref[...] = h0
    p0_ref[...] = dinv * h0
    c1_ref[...] = 0.9 / deg
    c1f_ref[...] = 0.9 * dinv


def _mlp_prep(x, W1, b1, W2, b2, sdeg):
    grid = N // MLP_BLK
    return pl.pallas_call(
        _mlp_body,
        grid=(grid,),
        in_specs=[
            pl.BlockSpec((MLP_BLK, FEATS), lambda i: (i, 0)),
            pl.BlockSpec((FEATS, HIDDEN), lambda i: (0, 0)),
            pl.BlockSpec((1, HIDDEN), lambda i: (0, 0)),
            pl.BlockSpec((HIDDEN, CLASSES), lambda i: (0, 0)),
            pl.BlockSpec((1, CLASSES), lambda i: (0, 0)),
            pl.BlockSpec((2, MLP_BLK, 32), lambda i: (0, i, 0)),
        ],
        out_specs=[
            pl.BlockSpec((MLP_BLK, CLASSES), lambda i: (i, 0)),
            pl.BlockSpec((MLP_BLK, CLASSES), lambda i: (i, 0)),
            pl.BlockSpec((MLP_BLK, 1), lambda i: (i, 0)),
            pl.BlockSpec((MLP_BLK, 1), lambda i: (i, 0)),
        ],
        out_shape=[
            jax.ShapeDtypeStruct((N, CLASSES), jnp.float32),
            jax.ShapeDtypeStruct((N, CLASSES), jnp.float32),
            jax.ShapeDtypeStruct((N, 1), jnp.float32),
            jax.ShapeDtypeStruct((N, 1), jnp.float32),
        ],
    )(x, W1, b1.reshape(1, HIDDEN), W2, b2.reshape(1, CLASSES), sdeg)


def _upd_body(s_ref, base_ref, coef_ref, out_ref):
    cf = coef_ref[...]
    out_ref[...] = (
        jnp.concatenate([cf * s_ref[0], cf * s_ref[1]], axis=1)
        + ALPHA * base_ref[...]
    )


def _upd(s, base, coef):
    grid = N // UPD_BLK
    return pl.pallas_call(
        _upd_body,
        grid=(grid,),
        in_specs=[
            pl.BlockSpec((2, UPD_BLK, 32), lambda i: (0, i, 0)),
            pl.BlockSpec((UPD_BLK, CLASSES), lambda i: (i, 0)),
            pl.BlockSpec((UPD_BLK, 1), lambda i: (i, 0)),
        ],
        out_specs=pl.BlockSpec((UPD_BLK, CLASSES), lambda i: (i, 0)),
        out_shape=jax.ShapeDtypeStruct((N, CLASSES), jnp.float32),
    )(s, base, coef)


def kernel(x, edges, W1, b1, W2, b2):
    src = edges[0]
    dst = edges[1]
    padn = jnp.full((EP - E,), N, dtype=jnp.int32)
    ep = jnp.stack([
        jnp.concatenate([src, padn]),
        jnp.concatenate([dst, padn]),
    ]).reshape(2, EPR, 128)
    s2 = jnp.concatenate(
        [src, jnp.zeros((EPP - E,), jnp.int32)]).reshape(EPRP, 128)
    d2 = jnp.concatenate(
        [dst, jnp.full((EPP - E,), N, jnp.int32)]).reshape(EPRP, 128)
    cidx = jnp.stack([
        jnp.stack([2 * s2, d2], axis=1),
        jnp.stack([2 * s2 + 1, d2], axis=1),
    ])
    cidx = jnp.pad(cidx, ((0, 0), (0, CIDX_ROWS - EPRP), (0, 0), (0, 0)))

    ones = jnp.ones((128, 32), jnp.float32)
    zeros = jnp.zeros((ZROWS, 32), jnp.float32)

    sdeg = _deg_sc(ep, ones, zeros)
    h0, p0, c1, c1f = _mlp_prep(x, W1, b1, W2, b2, sdeg)

    p = p0
    for it in range(DEPTH):
        s = _prop_sc(p.reshape(2 * N, 32), cidx, zeros)
        if it < DEPTH - 1:
            p = _upd(s, p0, c1)
        else:
            p = _upd(s, h0, c1f)
    return p


# speedup vs baseline: 13.1295x; 1.1017x over previous
import functools

import jax
import jax.numpy as jnp
from jax import lax
from jax.experimental import pallas as pl
from jax.experimental.pallas import tpu as pltpu
from jax.experimental.pallas import tpu_sc as plsc

N = 50000
FEATS = 128
HIDDEN = 64
CLASSES = 64
ALPHA = 0.1
DEPTH = 10

E = 800000
EP = 802816
EPR = EP // 128
ROWS_PER_TILE = EPR // 16
DEG_BROWS = 8
DEG_BLOCKS = ROWS_PER_TILE // DEG_BROWS

EPP = 804864
EPRP = EPP // 128
PROWS = EPRP // 16
PB = 3
PBLOCKS = PROWS // PB
CIDX_ROWS = EPRP + 8

NPAD = 50048
ZROWS = NPAD // 16
WROWS = 3128
NOUT = 16 * WROWS

MLP_BLK = 1000
UPD_BLK = 2000

_MESH = plsc.VectorSubcoreMesh(core_axis_name="c", subcore_axis_name="s")
_SC_PARAMS = pltpu.CompilerParams(use_tc_tiling_on_sc=False)


@functools.partial(
    pl.kernel,
    out_type=jax.ShapeDtypeStruct((2, NOUT, 32), jnp.float32),
    mesh=_MESH,
    scratch_types=[
        pltpu.VMEM_SHARED((NPAD, 32), jnp.float32),
        pltpu.VMEM((128, 32), jnp.float32),
        pltpu.VMEM((8, 128), jnp.int32),
        pltpu.SemaphoreType.DMA,
    ],
    compiler_params=_SC_PARAMS,
)
def _deg_sc(ep_hbm, ones_hbm, zeros_hbm, sdeg_hbm, acc, onesv, idxv, sem):
    c = lax.axis_index("c")
    t = lax.axis_index("s")
    pltpu.sync_copy(zeros_hbm, acc.at[pl.ds(t * ZROWS, ZROWS)])
    pltpu.sync_copy(ones_hbm, onesv)
    plsc.subcore_barrier()

    def blk(i, carry):
        bb = t * ROWS_PER_TILE + i * DEG_BROWS
        pltpu.sync_copy(ep_hbm.at[c, pl.ds(bb, DEG_BROWS)], idxv)
        cps = [
            pltpu.async_copy(onesv, acc.at[idxv.at[j]], sem, add=True)
            for j in range(DEG_BROWS)
        ]
        for d in cps:
            d.wait()
        return carry

    lax.fori_loop(0, DEG_BLOCKS, blk, 0)
    plsc.subcore_barrier()
    rb = t * WROWS
    pltpu.sync_copy(acc.at[pl.ds(rb, WROWS)], sdeg_hbm.at[c, pl.ds(rb, WROWS)])


@functools.partial(
    pl.kernel,
    out_type=jax.ShapeDtypeStruct((2, NOUT, 32), jnp.float32),
    mesh=_MESH,
    scratch_types=[
        pltpu.VMEM_SHARED((NPAD, 32), jnp.float32),
        pltpu.VMEM((3, PB, 2, 128), jnp.int32),
        pltpu.VMEM((2, PB * 128, 32), jnp.float32),
        pltpu.SemaphoreType.DMA,
        pltpu.SemaphoreType.DMA,
        pltpu.SemaphoreType.DMA,
    ],
    compiler_params=_SC_PARAMS,
)
def _prop_sc(p2_hbm, cidx_hbm, zeros_hbm, s_hbm,
             acc, ib, rows, sem_i, sem_g, sem_s):
    c = lax.axis_index("c")
    t = lax.axis_index("s")
    pltpu.sync_copy(zeros_hbm, acc.at[pl.ds(t * ZROWS, ZROWS)])
    base = t * PROWS
    pltpu.sync_copy(cidx_hbm.at[c, pl.ds(base, PB)], ib.at[0])
    pltpu.async_copy(cidx_hbm.at[c, pl.ds(base + PB, PB)], ib.at[1], sem_i)
    plsc.subcore_barrier()
    for j in range(PB):
        pltpu.async_copy(p2_hbm.at[ib.at[0, j, 0]],
                         rows.at[0, pl.ds(j * 128, 128)], sem_g)

    def blk(k, carry):
        cur3 = k % 3
        nxt3 = (k + 1) % 3
        ovr3 = (k + 2) % 3
        cur2 = k % 2
        nxt2 = (k + 1) % 2

        @pl.when(k > 0)
        def _():
            for j in range(PB):
                pltpu.make_async_copy(rows.at[nxt2, pl.ds(j * 128, 128)],
                                      acc.at[ib.at[cur3, j, 1]], sem_s).wait()

        @pl.when(k < PBLOCKS - 2)
        def _():
            pltpu.async_copy(cidx_hbm.at[c, pl.ds(base + (k + 2) * PB, PB)],
                             ib.at[ovr3], sem_i)

        @pl.when(k < PBLOCKS - 1)
        def _():
            pltpu.make_async_copy(cidx_hbm.at[c, pl.ds(base, PB)],
                                  ib.at[nxt3], sem_i).wait()
            for j in range(PB):
                pltpu.async_copy(p2_hbm.at[ib.at[nxt3, j, 0]],
                                 rows.at[nxt2, pl.ds(j * 128, 128)], sem_g)

        for j in range(PB):
            pltpu.make_async_copy(p2_hbm.at[ib.at[cur3, j, 0]],
                                  rows.at[cur2, pl.ds(j * 128, 128)],
                                  sem_g).wait()
        for j in range(PB):
            pltpu.async_copy(rows.at[cur2, pl.ds(j * 128, 128)],
                             acc.at[ib.at[cur3, j, 1]], sem_s, add=True)
        return carry

    lax.fori_loop(0, PBLOCKS, blk, 0)
    for j in range(PB):
        pltpu.make_async_copy(rows.at[0, pl.ds(j * 128, 128)],
                              acc.at[ib.at[1, j, 1]], sem_s).wait()
    plsc.subcore_barrier()
    rb = t * WROWS
    pltpu.sync_copy(acc.at[pl.ds(rb, WROWS)], s_hbm.at[c, pl.ds(rb, WROWS)])


def _mlp_body(x_ref, w1_ref, b1_ref, w2_ref, b2_ref, sd_ref,
              h0_ref, p0_ref, c1_ref, c1f_ref):
    deg = jnp.maximum(sd_ref[0, :, 0:1] + sd_ref[1, :, 0:1], 1.0)
    dinv = lax.rsqrt(deg)
    h = jnp.maximum(
        jnp.dot(x_ref[...], w1_ref[...], preferred_element_type=jnp.float32)
        + b1_ref[...],
        0.0,
    )
    h0 = jnp.dot(h, w2_ref[...], preferred_element_type=jnp.float32) + b2_ref[...]
    h0_ref[...] = h0
    p0_ref[...] = dinv * h0
    c1_ref[...] = 0.9 / deg
    c1f_ref[...] = 0.9 * dinv


def _mlp_prep(x, W1, b1, W2, b2, sdeg):
    grid = N // MLP_BLK
    return pl.pallas_call(
        _mlp_body,
        grid=(grid,),
        in_specs=[
            pl.BlockSpec((MLP_BLK, FEATS), lambda i: (i, 0)),
            pl.BlockSpec((FEATS, HIDDEN), lambda i: (0, 0)),
            pl.BlockSpec((1, HIDDEN), lambda i: (0, 0)),
            pl.BlockSpec((HIDDEN, CLASSES), lambda i: (0, 0)),
            pl.BlockSpec((1, CLASSES), lambda i: (0, 0)),
            pl.BlockSpec((2, MLP_BLK, 32), lambda i: (0, i, 0)),
        ],
        out_specs=[
            pl.BlockSpec((MLP_BLK, CLASSES), lambda i: (i, 0)),
            pl.BlockSpec((MLP_BLK, CLASSES), lambda i: (i, 0)),
            pl.BlockSpec((MLP_BLK, 1), lambda i: (i, 0)),
            pl.BlockSpec((MLP_BLK, 1), lambda i: (i, 0)),
        ],
        out_shape=[
            jax.ShapeDtypeStruct((N, CLASSES), jnp.float32),
            jax.ShapeDtypeStruct((N, CLASSES), jnp.float32),
            jax.ShapeDtypeStruct((N, 1), jnp.float32),
            jax.ShapeDtypeStruct((N, 1), jnp.float32),
        ],
    )(x, W1, b1.reshape(1, HIDDEN), W2, b2.reshape(1, CLASSES), sdeg)


def _upd_body(s_ref, base_ref, coef_ref, out_ref):
    cf = coef_ref[...]
    out_ref[...] = (
        jnp.concatenate([cf * s_ref[0], cf * s_ref[1]], axis=1)
        + ALPHA * base_ref[...]
    )


def _upd(s, base, coef):
    grid = N // UPD_BLK
    return pl.pallas_call(
        _upd_body,
        grid=(grid,),
        in_specs=[
            pl.BlockSpec((2, UPD_BLK, 32), lambda i: (0, i, 0)),
            pl.BlockSpec((UPD_BLK, CLASSES), lambda i: (i, 0)),
            pl.BlockSpec((UPD_BLK, 1), lambda i: (i, 0)),
        ],
        out_specs=pl.BlockSpec((UPD_BLK, CLASSES), lambda i: (i, 0)),
        out_shape=jax.ShapeDtypeStruct((N, CLASSES), jnp.float32),
    )(s, base, coef)


def kernel(x, edges, W1, b1, W2, b2):
    src = edges[0]
    dst = edges[1]
    padn = jnp.full((EP - E,), N, dtype=jnp.int32)
    ep = jnp.stack([
        jnp.concatenate([src, padn]),
        jnp.concatenate([dst, padn]),
    ]).reshape(2, EPR, 128)
    s2 = jnp.concatenate(
        [src, jnp.zeros((EPP - E,), jnp.int32)]).reshape(EPRP, 128)
    d2 = jnp.concatenate(
        [dst, jnp.full((EPP - E,), N, jnp.int32)]).reshape(EPRP, 128)
    cidx = jnp.stack([
        jnp.stack([2 * s2, d2], axis=1),
        jnp.stack([2 * s2 + 1, d2], axis=1),
    ])
    cidx = jnp.pad(cidx, ((0, 0), (0, CIDX_ROWS - EPRP), (0, 0), (0, 0)))

    ones = jnp.ones((128, 32), jnp.float32)
    zeros = jnp.zeros((ZROWS, 32), jnp.float32)

    sdeg = _deg_sc(ep, ones, zeros)
    h0, p0, c1, c1f = _mlp_prep(x, W1, b1, W2, b2, sdeg)

    p = p0
    for it in range(DEPTH):
        s = _prop_sc(p.reshape(2 * N, 32), cidx, zeros)
        if it < DEPTH - 1:
            p = _upd(s, p0, c1)
        else:
            p = _upd(s, h0, c1f)
    return p


# speedup vs baseline: 13.1987x; 1.0053x over previous
import functools

import jax
import jax.numpy as jnp
from jax import lax
from jax.experimental import pallas as pl
from jax.experimental.pallas import tpu as pltpu
from jax.experimental.pallas import tpu_sc as plsc

N = 50000
FEATS = 128
HIDDEN = 64
CLASSES = 64
ALPHA = 0.1
DEPTH = 10

E = 800000
EP = 802816
EPR = EP // 128
ROWS_PER_TILE = EPR // 16
DEG_BROWS = 8
DEG_BLOCKS = ROWS_PER_TILE // DEG_BROWS

EPP = 804864
EPRP = EPP // 128
PROWS = EPRP // 16
PB = 3
PBLOCKS = PROWS // PB
CIDX_ROWS = EPRP + 8

NPAD = 50048
ZROWS = NPAD // 16
WROWS = 3128
NOUT = 16 * WROWS

MLP_BLK = 1000
UPD_BLK = 2000

_MESH = plsc.VectorSubcoreMesh(core_axis_name="c", subcore_axis_name="s")
_SC_PARAMS = pltpu.CompilerParams(use_tc_tiling_on_sc=False)


@functools.partial(
    pl.kernel,
    out_type=jax.ShapeDtypeStruct((2, NOUT, 32), jnp.float32),
    mesh=_MESH,
    scratch_types=[
        pltpu.VMEM_SHARED((NPAD, 32), jnp.float32),
        pltpu.VMEM((128, 32), jnp.float32),
        pltpu.VMEM((3, DEG_BROWS, 128), jnp.int32),
        pltpu.SemaphoreType.DMA,
        pltpu.SemaphoreType.DMA,
    ],
    compiler_params=_SC_PARAMS,
)
def _deg_sc(ep_hbm, ones_hbm, zeros_hbm, sdeg_hbm, acc, onesv, idxv,
            sem_i, sem_s):
    c = lax.axis_index("c")
    t = lax.axis_index("s")
    base = t * ROWS_PER_TILE
    pltpu.sync_copy(zeros_hbm, acc.at[pl.ds(t * ZROWS, ZROWS)])
    pltpu.sync_copy(ones_hbm, onesv)
    pltpu.sync_copy(ep_hbm.at[c, pl.ds(base, DEG_BROWS)], idxv.at[0])
    plsc.subcore_barrier()

    def blk(k, carry):
        cur3 = k % 3
        nxt3 = (k + 1) % 3

        @pl.when(k > 0)
        def _():
            for j in range(DEG_BROWS):
                pltpu.make_async_copy(onesv, acc.at[idxv.at[cur3, j]],
                                      sem_s).wait()

        @pl.when(k < DEG_BLOCKS - 1)
        def _():
            pltpu.async_copy(
                ep_hbm.at[c, pl.ds(base + (k + 1) * DEG_BROWS, DEG_BROWS)],
                idxv.at[nxt3], sem_i)

        @pl.when(k > 0)
        def _():
            pltpu.make_async_copy(ep_hbm.at[c, pl.ds(base, DEG_BROWS)],
                                  idxv.at[cur3], sem_i).wait()

        for j in range(DEG_BROWS):
            pltpu.async_copy(onesv, acc.at[idxv.at[cur3, j]], sem_s, add=True)
        return carry

    lax.fori_loop(0, DEG_BLOCKS, blk, 0)
    for j in range(DEG_BROWS):
        pltpu.make_async_copy(onesv, acc.at[idxv.at[0, j]], sem_s).wait()
    plsc.subcore_barrier()
    rb = t * WROWS
    pltpu.sync_copy(acc.at[pl.ds(rb, WROWS)], sdeg_hbm.at[c, pl.ds(rb, WROWS)])


@functools.partial(
    pl.kernel,
    out_type=jax.ShapeDtypeStruct((2, NOUT, 32), jnp.float32),
    mesh=_MESH,
    scratch_types=[
        pltpu.VMEM_SHARED((NPAD, 32), jnp.float32),
        pltpu.VMEM((3, PB, 2, 128), jnp.int32),
        pltpu.VMEM((2, PB * 128, 32), jnp.float32),
        pltpu.SemaphoreType.DMA,
        pltpu.SemaphoreType.DMA,
        pltpu.SemaphoreType.DMA,
    ],
    compiler_params=_SC_PARAMS,
)
def _prop_sc(p2_hbm, cidx_hbm, zeros_hbm, s_hbm,
             acc, ib, rows, sem_i, sem_g, sem_s):
    c = lax.axis_index("c")
    t = lax.axis_index("s")
    pltpu.sync_copy(zeros_hbm, acc.at[pl.ds(t * ZROWS, ZROWS)])
    base = t * PROWS
    pltpu.sync_copy(cidx_hbm.at[c, pl.ds(base, PB)], ib.at[0])
    pltpu.async_copy(cidx_hbm.at[c, pl.ds(base + PB, PB)], ib.at[1], sem_i)
    plsc.subcore_barrier()
    for j in range(PB):
        pltpu.async_copy(p2_hbm.at[ib.at[0, j, 0]],
                         rows.at[0, pl.ds(j * 128, 128)], sem_g)

    def blk(k, carry):
        cur3 = k % 3
        nxt3 = (k + 1) % 3
        ovr3 = (k + 2) % 3
        cur2 = k % 2
        nxt2 = (k + 1) % 2

        @pl.when(k > 0)
        def _():
            for j in range(PB):
                pltpu.make_async_copy(rows.at[nxt2, pl.ds(j * 128, 128)],
                                      acc.at[ib.at[cur3, j, 1]], sem_s).wait()

        @pl.when(k < PBLOCKS - 2)
        def _():
            pltpu.async_copy(cidx_hbm.at[c, pl.ds(base + (k + 2) * PB, PB)],
                             ib.at[ovr3], sem_i)

        @pl.when(k < PBLOCKS - 1)
        def _():
            pltpu.make_async_copy(cidx_hbm.at[c, pl.ds(base, PB)],
                                  ib.at[nxt3], sem_i).wait()
            for j in range(PB):
                pltpu.async_copy(p2_hbm.at[ib.at[nxt3, j, 0]],
                                 rows.at[nxt2, pl.ds(j * 128, 128)], sem_g)

        for j in range(PB):
            pltpu.make_async_copy(p2_hbm.at[ib.at[cur3, j, 0]],
                                  rows.at[cur2, pl.ds(j * 128, 128)],
                                  sem_g).wait()
        for j in range(PB):
            pltpu.async_copy(rows.at[cur2, pl.ds(j * 128, 128)],
                             acc.at[ib.at[cur3, j, 1]], sem_s, add=True)
        return carry

    lax.fori_loop(0, PBLOCKS, blk, 0)
    for j in range(PB):
        pltpu.make_async_copy(rows.at[0, pl.ds(j * 128, 128)],
                              acc.at[ib.at[1, j, 1]], sem_s).wait()
    plsc.subcore_barrier()
    rb = t * WROWS
    pltpu.sync_copy(acc.at[pl.ds(rb, WROWS)], s_hbm.at[c, pl.ds(rb, WROWS)])


def _mlp_body(x_ref, w1_ref, b1_ref, w2_ref, b2_ref, sd_ref,
              h0_ref, p0_ref, c1_ref, c1f_ref):
    deg = jnp.maximum(sd_ref[0, :, 0:1] + sd_ref[1, :, 0:1], 1.0)
    dinv = lax.rsqrt(deg)
    h = jnp.maximum(
        jnp.dot(x_ref[...], w1_ref[...], preferred_element_type=jnp.float32)
        + b1_ref[...],
        0.0,
    )
    h0 = jnp.dot(h, w2_ref[...], preferred_element_type=jnp.float32) + b2_ref[...]
    h0_ref[...] = h0
    p0_ref[...] = dinv * h0
    c1_ref[...] = 0.9 / deg
    c1f_ref[...] = 0.9 * dinv


def _mlp_prep(x, W1, b1, W2, b2, sdeg):
    grid = N // MLP_BLK
    return pl.pallas_call(
        _mlp_body,
        grid=(grid,),
        in_specs=[
            pl.BlockSpec((MLP_BLK, FEATS), lambda i: (i, 0)),
            pl.BlockSpec((FEATS, HIDDEN), lambda i: (0, 0)),
            pl.BlockSpec((1, HIDDEN), lambda i: (0, 0)),
            pl.BlockSpec((HIDDEN, CLASSES), lambda i: (0, 0)),
            pl.BlockSpec((1, CLASSES), lambda i: (0, 0)),
            pl.BlockSpec((2, MLP_BLK, 32), lambda i: (0, i, 0)),
        ],
        out_specs=[
            pl.BlockSpec((MLP_BLK, CLASSES), lambda i: (i, 0)),
            pl.BlockSpec((MLP_BLK, CLASSES), lambda i: (i, 0)),
            pl.BlockSpec((MLP_BLK, 1), lambda i: (i, 0)),
            pl.BlockSpec((MLP_BLK, 1), lambda i: (i, 0)),
        ],
        out_shape=[
            jax.ShapeDtypeStruct((N, CLASSES), jnp.float32),
            jax.ShapeDtypeStruct((N, CLASSES), jnp.float32),
            jax.ShapeDtypeStruct((N, 1), jnp.float32),
            jax.ShapeDtypeStruct((N, 1), jnp.float32),
        ],
    )(x, W1, b1.reshape(1, HIDDEN), W2, b2.reshape(1, CLASSES), sdeg)


def _upd_body(s_ref, base_ref, coef_ref, out_ref):
    cf = coef_ref[...]
    out_ref[...] = (
        jnp.concatenate([cf * s_ref[0], cf * s_ref[1]], axis=1)
        + ALPHA * base_ref[...]
    )


def _upd(s, base, coef):
    grid = N // UPD_BLK
    return pl.pallas_call(
        _upd_body,
        grid=(grid,),
        in_specs=[
            pl.BlockSpec((2, UPD_BLK, 32), lambda i: (0, i, 0)),
            pl.BlockSpec((UPD_BLK, CLASSES), lambda i: (i, 0)),
            pl.BlockSpec((UPD_BLK, 1), lambda i: (i, 0)),
        ],
        out_specs=pl.BlockSpec((UPD_BLK, CLASSES), lambda i: (i, 0)),
        out_shape=jax.ShapeDtypeStruct((N, CLASSES), jnp.float32),
    )(s, base, coef)


def kernel(x, edges, W1, b1, W2, b2):
    src = edges[0]
    dst = edges[1]
    padn = jnp.full((EP - E,), N, dtype=jnp.int32)
    ep = jnp.stack([
        jnp.concatenate([src, padn]),
        jnp.concatenate([dst, padn]),
    ]).reshape(2, EPR, 128)
    s2 = jnp.concatenate(
        [src, jnp.zeros((EPP - E,), jnp.int32)]).reshape(EPRP, 128)
    d2 = jnp.concatenate(
        [dst, jnp.full((EPP - E,), N, jnp.int32)]).reshape(EPRP, 128)
    cidx = jnp.stack([
        jnp.stack([2 * s2, d2], axis=1),
        jnp.stack([2 * s2 + 1, d2], axis=1),
    ])
    cidx = jnp.pad(cidx, ((0, 0), (0, CIDX_ROWS - EPRP), (0, 0), (0, 0)))

    ones = jnp.ones((128, 32), jnp.float32)
    zeros = jnp.zeros((ZROWS, 32), jnp.float32)

    sdeg = _deg_sc(ep, ones, zeros)
    h0, p0, c1, c1f = _mlp_prep(x, W1, b1, W2, b2, sdeg)

    p = p0
    for it in range(DEPTH):
        s = _prop_sc(p.reshape(2 * N, 32), cidx, zeros)
        if it < DEPTH - 1:
            p = _upd(s, p0, c1)
        else:
            p = _upd(s, h0, c1f)
    return p


# speedup vs baseline: 13.6711x; 1.0358x over previous
import functools

import jax
import jax.numpy as jnp
from jax import lax
from jax.experimental import pallas as pl
from jax.experimental.pallas import tpu as pltpu
from jax.experimental.pallas import tpu_sc as plsc

N = 50000
FEATS = 128
HIDDEN = 64
CLASSES = 64
ALPHA = 0.1
DEPTH = 10

E = 800000
EP = 802816
EPR = EP // 128
ROWS_PER_TILE = EPR // 16
DEG_BROWS = 8
DEG_BLOCKS = ROWS_PER_TILE // DEG_BROWS

EPP = 804864
EPRP = EPP // 128
PROWS = EPRP // 16
PB = 3
PBLOCKS = PROWS // PB
CIDX_ROWS = EPRP + 8

NPAD = 51200
ZROWS = NPAD // 16
WROWS = 3128
NOUT = 16 * WROWS
UROWS = ZROWS
WIDX_R = NPAD // 128
PD = 2 * NPAD

MLP_BLK = 1000
UPD_BLK = 2000

_MESH = plsc.VectorSubcoreMesh(core_axis_name="c", subcore_axis_name="s")
_SC_PARAMS = pltpu.CompilerParams(use_tc_tiling_on_sc=False)


@functools.partial(
    pl.kernel,
    out_type=jax.ShapeDtypeStruct((2, NOUT, 32), jnp.float32),
    mesh=_MESH,
    scratch_types=[
        pltpu.VMEM_SHARED((NPAD, 32), jnp.float32),
        pltpu.VMEM((128, 32), jnp.float32),
        pltpu.VMEM((3, DEG_BROWS, 128), jnp.int32),
        pltpu.SemaphoreType.DMA,
        pltpu.SemaphoreType.DMA,
    ],
    compiler_params=_SC_PARAMS,
)
def _deg_sc(ep_hbm, ones_hbm, zeros_hbm, sdeg_hbm, acc, onesv, idxv,
            sem_i, sem_s):
    c = lax.axis_index("c")
    t = lax.axis_index("s")
    base = t * ROWS_PER_TILE
    pltpu.sync_copy(zeros_hbm, acc.at[pl.ds(t * ZROWS, ZROWS)])
    pltpu.sync_copy(ones_hbm, onesv)
    pltpu.sync_copy(ep_hbm.at[c, pl.ds(base, DEG_BROWS)], idxv.at[0])
    plsc.subcore_barrier()

    def blk(k, carry):
        cur3 = k % 3
        nxt3 = (k + 1) % 3

        @pl.when(k > 0)
        def _():
            for j in range(DEG_BROWS):
                pltpu.make_async_copy(onesv, acc.at[idxv.at[cur3, j]],
                                      sem_s).wait()

        @pl.when(k < DEG_BLOCKS - 1)
        def _():
            pltpu.async_copy(
                ep_hbm.at[c, pl.ds(base + (k + 1) * DEG_BROWS, DEG_BROWS)],
                idxv.at[nxt3], sem_i)

        @pl.when(k > 0)
        def _():
            pltpu.make_async_copy(ep_hbm.at[c, pl.ds(base, DEG_BROWS)],
                                  idxv.at[cur3], sem_i).wait()

        for j in range(DEG_BROWS):
            pltpu.async_copy(onesv, acc.at[idxv.at[cur3, j]], sem_s, add=True)
        return carry

    lax.fori_loop(0, DEG_BLOCKS, blk, 0)
    for j in range(DEG_BROWS):
        pltpu.make_async_copy(onesv, acc.at[idxv.at[0, j]], sem_s).wait()
    plsc.subcore_barrier()
    rb = t * WROWS
    pltpu.sync_copy(acc.at[pl.ds(rb, WROWS)], sdeg_hbm.at[c, pl.ds(rb, WROWS)])


@functools.partial(
    pl.kernel,
    out_type=[
        jax.ShapeDtypeStruct((PD, 32), jnp.float32),
        jax.ShapeDtypeStruct((PD, 32), jnp.float32),
        jax.ShapeDtypeStruct((PD, 32), jnp.float32),
    ],
    mesh=_MESH,
    scratch_types=[
        pltpu.VMEM_SHARED((NPAD, 32), jnp.float32),
        pltpu.VMEM((3, PB, 2, 128), jnp.int32),
        pltpu.VMEM((2, PB * 128, 32), jnp.float32),
        pltpu.VMEM((1, 128), jnp.int32),
        pltpu.SemaphoreType.DMA,
        pltpu.SemaphoreType.DMA,
        pltpu.SemaphoreType.DMA,
    ],
    compiler_params=_SC_PARAMS,
)
def _appnp_sc(p0_hbm, cidx_hbm, widx_hbm, binit_hbm, binitf_hbm,
              c1x_hbm, c1fx_hbm, h_hbm, pa_hbm, pb_hbm,
              acc, ib, rows, uv, sem_i, sem_g, sem_s):
    c = lax.axis_index("c")
    t = lax.axis_index("s")
    v0 = t * UROWS
    base = t * PROWS

    def edge_blk_body(p_hbm, k):
        cur3 = k % 3
        nxt3 = (k + 1) % 3
        ovr3 = (k + 2) % 3
        cur2 = k % 2
        nxt2 = (k + 1) % 2

        @pl.when(k > 0)
        def _():
            for j in range(PB):
                pltpu.make_async_copy(rows.at[nxt2, pl.ds(j * 128, 128)],
                                      acc.at[ib.at[cur3, j, 1]], sem_s).wait()

        @pl.when(k < PBLOCKS - 2)
        def _():
            pltpu.async_copy(cidx_hbm.at[c, pl.ds(base + (k + 2) * PB, PB)],
                             ib.at[ovr3], sem_i)

        @pl.when(k < PBLOCKS - 1)
        def _():
            pltpu.make_async_copy(cidx_hbm.at[c, pl.ds(base, PB)],
                                  ib.at[nxt3], sem_i).wait()
            for j in range(PB):
                pltpu.async_copy(p_hbm.at[ib.at[nxt3, j, 0]],
                                 rows.at[nxt2, pl.ds(j * 128, 128)], sem_g)

        for j in range(PB):
            pltpu.make_async_copy(p_hbm.at[ib.at[cur3, j, 0]],
                                  rows.at[cur2, pl.ds(j * 128, 128)],
                                  sem_g).wait()
        for j in range(PB):
            pltpu.async_copy(rows.at[cur2, pl.ds(j * 128, 128)],
                             acc.at[ib.at[cur3, j, 1]], sem_s, add=True)

    def edge_phase(p_hbm):
        pltpu.sync_copy(cidx_hbm.at[c, pl.ds(base, PB)], ib.at[0])
        pltpu.async_copy(cidx_hbm.at[c, pl.ds(base + PB, PB)], ib.at[1],
                         sem_i)
        for j in range(PB):
            pltpu.async_copy(p_hbm.at[ib.at[0, j, 0]],
                             rows.at[0, pl.ds(j * 128, 128)], sem_g)

        def blk(k, carry):
            edge_blk_body(p_hbm, k)
            return carry

        lax.fori_loop(0, PBLOCKS, blk, 0)
        for j in range(PB):
            pltpu.make_async_copy(rows.at[0, pl.ds(j * 128, 128)],
                                  acc.at[ib.at[1, j, 1]], sem_s).wait()

    def upd_phase(cx_hbm, dst_hbm):
        def chunk(ch, carry):
            r0 = v0 + ch * 128
            pltpu.sync_copy(widx_hbm.at[c, pl.ds(t * 25 + ch, 1)], uv)
            pltpu.sync_copy(acc.at[pl.ds(r0, 128)],
                            rows.at[0, pl.ds(0, 128)])
            pltpu.sync_copy(cx_hbm.at[pl.ds(r0, 128)],
                            rows.at[0, pl.ds(128, 128)])

            def cbody(i, carry2):
                for u in range(4):
                    r = i * 4 + u
                    for k2 in range(2):
                        sl = pl.ds(k2 * 16, 16)
                        rows[0, 256 + r, sl] = (
                            rows[0, 128 + r, sl] * rows[0, r, sl])
                return carry2

            lax.fori_loop(0, 32, cbody, 0)
            pltpu.sync_copy(rows.at[0, pl.ds(256, 128)], dst_hbm.at[uv.at[0]])
            return carry

        lax.fori_loop(0, 25, chunk, 0)

    pltpu.sync_copy(binit_hbm.at[c, pl.ds(v0, UROWS)], acc.at[pl.ds(v0, UROWS)])
    plsc.subcore_barrier()
    for it in range(DEPTH):
        if it == 0:
            p_cur = p0_hbm
        else:
            p_cur = pa_hbm if it % 2 == 1 else pb_hbm
        edge_phase(p_cur)
        plsc.subcore_barrier()
        if it == DEPTH - 1:
            upd_phase(c1fx_hbm, h_hbm)
        else:
            upd_phase(c1x_hbm, pa_hbm if it % 2 == 0 else pb_hbm)
        plsc.subcore_barrier()
        if it < DEPTH - 1:
            nb = binit_hbm if it < DEPTH - 2 else binitf_hbm
            pltpu.sync_copy(nb.at[c, pl.ds(v0, UROWS)],
                            acc.at[pl.ds(v0, UROWS)])
            plsc.subcore_barrier()


def _mlp_body(x_ref, w1_ref, b1_ref, w2_ref, b2_ref, sd_ref,
              h0_ref, p0_ref, c1_ref, c1f_ref):
    deg = jnp.maximum(sd_ref[0, :, 0:1] + sd_ref[1, :, 0:1], 1.0)
    dinv = lax.rsqrt(deg)
    h = jnp.maximum(
        jnp.dot(x_ref[...], w1_ref[...], preferred_element_type=jnp.float32)
        + b1_ref[...],
        0.0,
    )
    h0 = jnp.dot(h, w2_ref[...], preferred_element_type=jnp.float32) + b2_ref[...]
    h0_ref[...] = h0
    p0_ref[...] = dinv * h0
    c1_ref[...] = 0.9 / deg
    c1f_ref[...] = 0.9 * dinv


def _mlp_prep(x, W1, b1, W2, b2, sdeg):
    grid = N // MLP_BLK
    return pl.pallas_call(
        _mlp_body,
        grid=(grid,),
        in_specs=[
            pl.BlockSpec((MLP_BLK, FEATS), lambda i: (i, 0)),
            pl.BlockSpec((FEATS, HIDDEN), lambda i: (0, 0)),
            pl.BlockSpec((1, HIDDEN), lambda i: (0, 0)),
            pl.BlockSpec((HIDDEN, CLASSES), lambda i: (0, 0)),
            pl.BlockSpec((1, CLASSES), lambda i: (0, 0)),
            pl.BlockSpec((2, MLP_BLK, 32), lambda i: (0, i, 0)),
        ],
        out_specs=[
            pl.BlockSpec((MLP_BLK, CLASSES), lambda i: (i, 0)),
            pl.BlockSpec((MLP_BLK, CLASSES), lambda i: (i, 0)),
            pl.BlockSpec((MLP_BLK, 1), lambda i: (i, 0)),
            pl.BlockSpec((MLP_BLK, 1), lambda i: (i, 0)),
        ],
        out_shape=[
            jax.ShapeDtypeStruct((N, CLASSES), jnp.float32),
            jax.ShapeDtypeStruct((N, CLASSES), jnp.float32),
            jax.ShapeDtypeStruct((N, 1), jnp.float32),
            jax.ShapeDtypeStruct((N, 1), jnp.float32),
        ],
    )(x, W1, b1.reshape(1, HIDDEN), W2, b2.reshape(1, CLASSES), sdeg)


def kernel(x, edges, W1, b1, W2, b2):
    src = edges[0]
    dst = edges[1]
    padn = jnp.full((EP - E,), N, dtype=jnp.int32)
    ep = jnp.stack([
        jnp.concatenate([src, padn]),
        jnp.concatenate([dst, padn]),
    ]).reshape(2, EPR, 128)
    s2 = jnp.concatenate(
        [src, jnp.zeros((EPP - E,), jnp.int32)]).reshape(EPRP, 128)
    d2 = jnp.concatenate(
        [dst, jnp.full((EPP - E,), N, jnp.int32)]).reshape(EPRP, 128)
    cidx = jnp.stack([
        jnp.stack([2 * s2, d2], axis=1),
        jnp.stack([2 * s2 + 1, d2], axis=1),
    ])
    cidx = jnp.pad(cidx, ((0, 0), (0, CIDX_ROWS - EPRP), (0, 0), (0, 0)))

    ones = jnp.ones((128, 32), jnp.float32)
    zeros = jnp.zeros((ZROWS, 32), jnp.float32)

    sdeg = _deg_sc(ep, ones, zeros)
    h0, p0, c1, c1f = _mlp_prep(x, W1, b1, W2, b2, sdeg)

    l = jnp.arange(NPAD, dtype=jnp.int32)
    widx = jnp.stack([(2 * l).reshape(WIDX_R, 128),
                      (2 * l + 1).reshape(WIDX_R, 128)])
    rpad = NPAD - N
    c1xP = jnp.pad(jnp.broadcast_to(c1, (N, 32)), ((0, rpad), (0, 0)))
    c1fxP = jnp.pad(jnp.broadcast_to(c1f, (N, 32)), ((0, rpad), (0, 0)))
    binit = (ALPHA * p0) / c1
    binitM = jnp.pad(jnp.stack([binit[:, :32], binit[:, 32:]]),
                     ((0, 0), (0, rpad), (0, 0)))
    binitf = (ALPHA * h0) / c1f
    binitfM = jnp.pad(jnp.stack([binitf[:, :32], binitf[:, 32:]]),
                      ((0, 0), (0, rpad), (0, 0)))

    h_pad, _, _ = _appnp_sc(p0.reshape(2 * N, 32), cidx, widx,
                            binitM, binitfM, c1xP, c1fxP)
    return h_pad[:2 * N].reshape(N, CLASSES)


# speedup vs baseline: 14.7230x; 1.0769x over previous
import functools

import jax
import jax.numpy as jnp
from jax import lax
from jax.experimental import pallas as pl
from jax.experimental.pallas import tpu as pltpu
from jax.experimental.pallas import tpu_sc as plsc

N = 50000
FEATS = 128
HIDDEN = 64
CLASSES = 64
ALPHA = 0.1
DEPTH = 10

E = 800000
EP = 802816
EPR = EP // 128
ROWS_PER_TILE = EPR // 16
DEG_BROWS = 8
DEG_BLOCKS = ROWS_PER_TILE // DEG_BROWS

EPP = 804864
EPRP = EPP // 128
PROWS = EPRP // 16
PB = 3
PBLOCKS = PROWS // PB
CIDX_ROWS = EPRP + 8

NPAD = 51200
ZROWS = NPAD // 16
WROWS = 3128
NOUT = 16 * WROWS
UROWS = ZROWS
WIDX_R = NPAD // 128
PD = 2 * NPAD

MLP_BLK = 1000
UPD_BLK = 2000

_MESH = plsc.VectorSubcoreMesh(core_axis_name="c", subcore_axis_name="s")
_SC_PARAMS = pltpu.CompilerParams(use_tc_tiling_on_sc=False)


@functools.partial(
    pl.kernel,
    out_type=jax.ShapeDtypeStruct((2, NOUT, 32), jnp.float32),
    mesh=_MESH,
    scratch_types=[
        pltpu.VMEM_SHARED((NPAD, 32), jnp.float32),
        pltpu.VMEM((128, 32), jnp.float32),
        pltpu.VMEM((3, DEG_BROWS, 128), jnp.int32),
        pltpu.SemaphoreType.DMA,
        pltpu.SemaphoreType.DMA,
    ],
    compiler_params=_SC_PARAMS,
)
def _deg_sc(ep_hbm, ones_hbm, zeros_hbm, sdeg_hbm, acc, onesv, idxv,
            sem_i, sem_s):
    c = lax.axis_index("c")
    t = lax.axis_index("s")
    base = t * ROWS_PER_TILE
    pltpu.sync_copy(zeros_hbm, acc.at[pl.ds(t * ZROWS, ZROWS)])
    pltpu.sync_copy(ones_hbm, onesv)
    pltpu.sync_copy(ep_hbm.at[c, pl.ds(base, DEG_BROWS)], idxv.at[0])
    plsc.subcore_barrier()

    def blk(k, carry):
        cur3 = k % 3
        nxt3 = (k + 1) % 3

        @pl.when(k > 0)
        def _():
            for j in range(DEG_BROWS):
                pltpu.make_async_copy(onesv, acc.at[idxv.at[cur3, j]],
                                      sem_s).wait()

        @pl.when(k < DEG_BLOCKS - 1)
        def _():
            pltpu.async_copy(
                ep_hbm.at[c, pl.ds(base + (k + 1) * DEG_BROWS, DEG_BROWS)],
                idxv.at[nxt3], sem_i)

        @pl.when(k > 0)
        def _():
            pltpu.make_async_copy(ep_hbm.at[c, pl.ds(base, DEG_BROWS)],
                                  idxv.at[cur3], sem_i).wait()

        for j in range(DEG_BROWS):
            pltpu.async_copy(onesv, acc.at[idxv.at[cur3, j]], sem_s, add=True)
        return carry

    lax.fori_loop(0, DEG_BLOCKS, blk, 0)
    for j in range(DEG_BROWS):
        pltpu.make_async_copy(onesv, acc.at[idxv.at[0, j]], sem_s).wait()
    plsc.subcore_barrier()
    rb = t * WROWS
    pltpu.sync_copy(acc.at[pl.ds(rb, WROWS)], sdeg_hbm.at[c, pl.ds(rb, WROWS)])


@functools.partial(
    pl.kernel,
    out_type=[
        jax.ShapeDtypeStruct((PD, 32), jnp.float32),
        jax.ShapeDtypeStruct((PD, 32), jnp.float32),
        jax.ShapeDtypeStruct((PD, 32), jnp.float32),
    ],
    mesh=_MESH,
    scratch_types=[
        pltpu.VMEM_SHARED((NPAD, 32), jnp.float32),
        pltpu.VMEM((3, PB, 2, 128), jnp.int32),
        pltpu.VMEM((2, PB * 128, 32), jnp.float32),
        pltpu.VMEM((1, 128), jnp.int32),
        pltpu.SemaphoreType.DMA,
        pltpu.SemaphoreType.DMA,
        pltpu.SemaphoreType.DMA,
    ],
    compiler_params=_SC_PARAMS,
)
def _appnp_sc(p0_hbm, cidx_hbm, widx_hbm, binit_hbm, binitf_hbm,
              c1x_hbm, c1fx_hbm, h_hbm, pa_hbm, pb_hbm,
              acc, ib, rows, uv, sem_i, sem_g, sem_s):
    c = lax.axis_index("c")
    t = lax.axis_index("s")
    v0 = t * UROWS
    base = t * PROWS

    def edge_blk_body(p_hbm, k):
        cur3 = k % 3
        nxt3 = (k + 1) % 3
        ovr3 = (k + 2) % 3
        cur2 = k % 2
        nxt2 = (k + 1) % 2

        @pl.when(k > 0)
        def _():
            for j in range(PB):
                pltpu.make_async_copy(rows.at[nxt2, pl.ds(j * 128, 128)],
                                      acc.at[ib.at[cur3, j, 1]], sem_s).wait()

        @pl.when(k < PBLOCKS - 2)
        def _():
            pltpu.async_copy(cidx_hbm.at[c, pl.ds(base + (k + 2) * PB, PB)],
                             ib.at[ovr3], sem_i)

        @pl.when(k < PBLOCKS - 1)
        def _():
            pltpu.make_async_copy(cidx_hbm.at[c, pl.ds(base, PB)],
                                  ib.at[nxt3], sem_i).wait()
            for j in range(PB):
                pltpu.async_copy(p_hbm.at[ib.at[nxt3, j, 0]],
                                 rows.at[nxt2, pl.ds(j * 128, 128)], sem_g)

        for j in range(PB):
            pltpu.make_async_copy(p_hbm.at[ib.at[cur3, j, 0]],
                                  rows.at[cur2, pl.ds(j * 128, 128)],
                                  sem_g).wait()
        for j in range(PB):
            pltpu.async_copy(rows.at[cur2, pl.ds(j * 128, 128)],
                             acc.at[ib.at[cur3, j, 1]], sem_s, add=True)

    def edge_phase(p_hbm):
        pltpu.sync_copy(cidx_hbm.at[c, pl.ds(base, PB)], ib.at[0])
        pltpu.async_copy(cidx_hbm.at[c, pl.ds(base + PB, PB)], ib.at[1],
                         sem_i)
        for j in range(PB):
            pltpu.async_copy(p_hbm.at[ib.at[0, j, 0]],
                             rows.at[0, pl.ds(j * 128, 128)], sem_g)

        def blk(k, carry):
            edge_blk_body(p_hbm, k)
            return carry

        lax.fori_loop(0, PBLOCKS, blk, 0)
        for j in range(PB):
            pltpu.make_async_copy(rows.at[0, pl.ds(j * 128, 128)],
                                  acc.at[ib.at[1, j, 1]], sem_s).wait()

    def upd_phase(cx_hbm, dst_hbm):
        def chunk(ch, carry):
            r0 = v0 + ch * 128
            pltpu.sync_copy(widx_hbm.at[c, pl.ds(t * 25 + ch, 1)], uv)
            pltpu.sync_copy(acc.at[pl.ds(r0, 128)],
                            rows.at[0, pl.ds(0, 128)])
            pltpu.sync_copy(cx_hbm.at[pl.ds(r0, 128)],
                            rows.at[0, pl.ds(128, 128)])

            def cbody(i, carry2):
                for u in range(4):
                    r = i * 4 + u
                    for k2 in range(2):
                        sl = pl.ds(k2 * 16, 16)
                        rows[0, 256 + r, sl] = (
                            rows[0, 128 + r, sl] * rows[0, r, sl])
                return carry2

            lax.fori_loop(0, 32, cbody, 0)
            pltpu.sync_copy(rows.at[0, pl.ds(256, 128)], dst_hbm.at[uv.at[0]])
            return carry

        lax.fori_loop(0, 25, chunk, 0)

    pltpu.sync_copy(binit_hbm.at[c, pl.ds(v0, UROWS)], acc.at[pl.ds(v0, UROWS)])
    plsc.subcore_barrier()
    for it in range(DEPTH):
        if it == 0:
            p_cur = p0_hbm
        else:
            p_cur = pa_hbm if it % 2 == 1 else pb_hbm
        edge_phase(p_cur)
        plsc.subcore_barrier()
        if it == DEPTH - 1:
            upd_phase(c1fx_hbm, h_hbm)
        else:
            upd_phase(c1x_hbm, pa_hbm if it % 2 == 0 else pb_hbm)
        plsc.subcore_barrier()
        if it < DEPTH - 1:
            nb = binit_hbm if it < DEPTH - 2 else binitf_hbm
            pltpu.sync_copy(nb.at[c, pl.ds(v0, UROWS)],
                            acc.at[pl.ds(v0, UROWS)])
            plsc.subcore_barrier()


def _mlp_body(x_ref, w1_ref, b1_ref, w2_ref, b2_ref, sd_ref,
              p0_ref, c1x_ref, c1fx_ref, bm_ref, bf_ref):
    deg = jnp.maximum(sd_ref[0, :, 0:1] + sd_ref[1, :, 0:1], 1.0)
    dinv = lax.rsqrt(deg)
    h = jnp.maximum(
        jnp.dot(x_ref[...], w1_ref[...], preferred_element_type=jnp.float32)
        + b1_ref[...],
        0.0,
    )
    h0 = jnp.dot(h, w2_ref[...], preferred_element_type=jnp.float32) + b2_ref[...]
    p0 = dinv * h0
    p0_ref[...] = p0
    c1x_ref[...] = jnp.broadcast_to(0.9 / deg, p0.shape[:1] + (32,))
    c1fx_ref[...] = jnp.broadcast_to(0.9 * dinv, p0.shape[:1] + (32,))
    binit = (ALPHA / 0.9) * p0 * deg
    bm_ref[...] = jnp.stack([binit[:, :32], binit[:, 32:]])
    binitf = (ALPHA / 0.9) * h0 / dinv
    bf_ref[...] = jnp.stack([binitf[:, :32], binitf[:, 32:]])


def _mlp_prep(x, W1, b1, W2, b2, sdeg):
    grid = N // MLP_BLK
    return pl.pallas_call(
        _mlp_body,
        grid=(grid,),
        in_specs=[
            pl.BlockSpec((MLP_BLK, FEATS), lambda i: (i, 0)),
            pl.BlockSpec((FEATS, HIDDEN), lambda i: (0, 0)),
            pl.BlockSpec((1, HIDDEN), lambda i: (0, 0)),
            pl.BlockSpec((HIDDEN, CLASSES), lambda i: (0, 0)),
            pl.BlockSpec((1, CLASSES), lambda i: (0, 0)),
            pl.BlockSpec((2, MLP_BLK, 32), lambda i: (0, i, 0)),
        ],
        out_specs=[
            pl.BlockSpec((MLP_BLK, CLASSES), lambda i: (i, 0)),
            pl.BlockSpec((MLP_BLK, 32), lambda i: (i, 0)),
            pl.BlockSpec((MLP_BLK, 32), lambda i: (i, 0)),
            pl.BlockSpec((2, MLP_BLK, 32), lambda i: (0, i, 0)),
            pl.BlockSpec((2, MLP_BLK, 32), lambda i: (0, i, 0)),
        ],
        out_shape=[
            jax.ShapeDtypeStruct((N, CLASSES), jnp.float32),
            jax.ShapeDtypeStruct((NPAD, 32), jnp.float32),
            jax.ShapeDtypeStruct((NPAD, 32), jnp.float32),
            jax.ShapeDtypeStruct((2, NPAD, 32), jnp.float32),
            jax.ShapeDtypeStruct((2, NPAD, 32), jnp.float32),
        ],
    )(x, W1, b1.reshape(1, HIDDEN), W2, b2.reshape(1, CLASSES), sdeg)


def kernel(x, edges, W1, b1, W2, b2):
    src = edges[0]
    dst = edges[1]
    padn = jnp.full((EP - E,), N, dtype=jnp.int32)
    ep = jnp.stack([
        jnp.concatenate([src, padn]),
        jnp.concatenate([dst, padn]),
    ]).reshape(2, EPR, 128)
    s2 = jnp.concatenate(
        [src, jnp.zeros((EPP - E,), jnp.int32)]).reshape(EPRP, 128)
    d2 = jnp.concatenate(
        [dst, jnp.full((EPP - E,), N, jnp.int32)]).reshape(EPRP, 128)
    cidx = jnp.stack([
        jnp.stack([2 * s2, d2], axis=1),
        jnp.stack([2 * s2 + 1, d2], axis=1),
    ])
    cidx = jnp.pad(cidx, ((0, 0), (0, CIDX_ROWS - EPRP), (0, 0), (0, 0)))

    ones = jnp.ones((128, 32), jnp.float32)
    zeros = jnp.zeros((ZROWS, 32), jnp.float32)

    l = jnp.arange(NPAD, dtype=jnp.int32)
    widx = jnp.stack([(2 * l).reshape(WIDX_R, 128),
                      (2 * l + 1).reshape(WIDX_R, 128)])

    sdeg = _deg_sc(ep, ones, zeros)
    p0, c1xP, c1fxP, binitM, binitfM = _mlp_prep(x, W1, b1, W2, b2, sdeg)

    h_pad, _, _ = _appnp_sc(p0.reshape(2 * N, 32), cidx, widx,
                            binitM, binitfM, c1xP, c1fxP)
    return h_pad[:2 * N].reshape(N, CLASSES)
